# baseline (speedup 1.0000x reference)
import functools

import jax
import jax.numpy as jnp
from jax import lax
from jax.experimental import pallas as pl
from jax.experimental.pallas import tpu as pltpu

GENE = 10000
B = 4096
D1, D2, D3 = 1024, 512, 256
K = 8192
COM_COST = 0.25

BM = 256
N_BT = B // BM
BK = 2048
NK = 5
BN = 2048
NJ = 5


def _enc_body(x_ref, w1_ref, b1_ref, w2_ref, b2_ref, w3_ref, b3_ref,
              z_ref, acc_ref):
    k = pl.program_id(0)
    i = pl.program_id(1)
    rem = GENE - k * BK
    x = x_ref[...]
    lanes = lax.broadcasted_iota(jnp.int32, (BM, BK), 1)
    x = jnp.where(lanes < rem, x, 0.0)
    w1 = w1_ref[...]
    rows = lax.broadcasted_iota(jnp.int32, (BK, D1), 0)
    w1 = jnp.where(rows < rem, w1, 0.0)
    part = jnp.dot(x, w1, preferred_element_type=jnp.float32)
    base = i * BM

    @pl.when(k == 0)
    def _():
        acc_ref[pl.ds(base, BM), :] = part

    @pl.when(k > 0)
    def _():
        acc_ref[pl.ds(base, BM), :] += part

    @pl.when(k == NK - 1)
    def _():
        z1 = jnp.maximum(acc_ref[pl.ds(base, BM), :] + b1_ref[...], 0.0)
        z2 = jnp.maximum(
            jnp.dot(z1, w2_ref[...], preferred_element_type=jnp.float32)
            + b2_ref[...], 0.0)
        z3 = jnp.maximum(
            jnp.dot(z2, w3_ref[...], preferred_element_type=jnp.float32)
            + b3_ref[...], 0.0)
        z_ref[...] = z3


def _vq_body(z_ref, c_ref, idx_ref):
    z = z_ref[...]
    c = c_ref[...]
    a = jnp.sum(z * z, axis=1, keepdims=True)
    bb = jnp.sum(c * c, axis=1)[None, :]
    zc = lax.dot_general(z, c, (((1,), (1,)), ((), ())),
                         preferred_element_type=jnp.float32)
    d = (a + bb) - 2.0 * zc
    m = jnp.min(d, axis=1, keepdims=True)
    ks = lax.broadcasted_iota(jnp.int32, d.shape, 1)
    idx = jnp.min(jnp.where(d == m, ks, K), axis=1).astype(jnp.int32)
    idx_ref[0, 0, :] = idx


def _dec_body(z_ref, q_ref, w4_ref, b4_ref, w5_ref, b5_ref, w6_ref, b6_ref,
              qst_ref, xrec_ref, ls_ref, h_ref):
    j = pl.program_id(0)
    i = pl.program_id(1)
    z = z_ref[...]
    q = q_ref[...]
    qst = z + (q - z)
    qst_ref[...] = qst

    @pl.when(jnp.logical_and(j == 0, i == 0))
    def _():
        ls_ref[...] = jnp.zeros((1, 1), jnp.float32)

    @pl.when(j == 0)
    def _():
        diff = q - z
        ls_ref[...] += jnp.sum(diff * diff, axis=(0, 1), keepdims=True)
        h1 = jnp.maximum(
            jnp.dot(qst, w4_ref[...], preferred_element_type=jnp.float32)
            + b4_ref[...], 0.0)
        h2 = jnp.maximum(
            jnp.dot(h1, w5_ref[...], preferred_element_type=jnp.float32)
            + b5_ref[...], 0.0)
        h_ref[pl.ds(i * BM, BM), :] = h2

    h = h_ref[pl.ds(i * BM, BM), :]
    xrec_ref[...] = (
        jnp.dot(h, w6_ref[...], preferred_element_type=jnp.float32)
        + b6_ref[...])


def kernel(inputs, W1, b1, W2, b2, W3, b3, codebook, W4, b4, W5, b5, W6, b6):
    b1r = b1.reshape(1, D1)
    b2r = b2.reshape(1, D2)
    b3r = b3.reshape(1, D3)
    b4r = b4.reshape(1, D2)
    b5r = b5.reshape(1, D1)
    b6r = b6.reshape(1, GENE)

    z = pl.pallas_call(
        _enc_body,
        grid=(NK, N_BT),
        in_specs=[
            pl.BlockSpec((BM, BK), lambda k, i: (i, k)),
            pl.BlockSpec((BK, D1), lambda k, i: (k, 0)),
            pl.BlockSpec((1, D1), lambda k, i: (0, 0)),
            pl.BlockSpec((D1, D2), lambda k, i: (0, 0)),
            pl.BlockSpec((1, D2), lambda k, i: (0, 0)),
            pl.BlockSpec((D2, D3), lambda k, i: (0, 0)),
            pl.BlockSpec((1, D3), lambda k, i: (0, 0)),
        ],
        out_specs=pl.BlockSpec((BM, D3), lambda k, i: (i, 0)),
        out_shape=jax.ShapeDtypeStruct((B, D3), jnp.float32),
        scratch_shapes=[pltpu.VMEM((B, D1), jnp.float32)],
    )(inputs, W1, b1r, W2, b2r, W3, b3r)

    idx3 = pl.pallas_call(
        _vq_body,
        grid=(N_BT,),
        in_specs=[
            pl.BlockSpec((BM, D3), lambda i: (i, 0)),
            pl.BlockSpec((K, D3), lambda i: (0, 0)),
        ],
        out_specs=pl.BlockSpec((1, 1, BM), lambda i: (i, 0, 0)),
        out_shape=jax.ShapeDtypeStruct((N_BT, 1, BM), jnp.int32),
    )(z, codebook)
    idx = idx3.reshape(B)

    quantized = jnp.take(codebook, idx, axis=0)

    qst, xrec, ls = pl.pallas_call(
        _dec_body,
        grid=(NJ, N_BT),
        in_specs=[
            pl.BlockSpec((BM, D3), lambda j, i: (i, 0)),
            pl.BlockSpec((BM, D3), lambda j, i: (i, 0)),
            pl.BlockSpec((D3, D2), lambda j, i: (0, 0)),
            pl.BlockSpec((1, D2), lambda j, i: (0, 0)),
            pl.BlockSpec((D2, D1), lambda j, i: (0, 0)),
            pl.BlockSpec((1, D1), lambda j, i: (0, 0)),
            pl.BlockSpec((D1, BN), lambda j, i: (0, j)),
            pl.BlockSpec((1, BN), lambda j, i: (0, j)),
        ],
        out_specs=[
            pl.BlockSpec((BM, D3), lambda j, i: (i, 0)),
            pl.BlockSpec((BM, BN), lambda j, i: (i, j)),
            pl.BlockSpec((1, 1), lambda j, i: (0, 0)),
        ],
        out_shape=[
            jax.ShapeDtypeStruct((B, D3), jnp.float32),
            jax.ShapeDtypeStruct((B, GENE), jnp.float32),
            jax.ShapeDtypeStruct((1, 1), jnp.float32),
        ],
        scratch_shapes=[pltpu.VMEM((B, D1), jnp.float32)],
    )(z, quantized, W4, b4r, W5, b5r, W6, b6r)

    mean_se = ls[0, 0] / (B * D3)
    loss = mean_se + COM_COST * mean_se
    return (loss, xrec, qst)
